# Initial kernel scaffold; baseline (speedup 1.0000x reference)
import jax
import jax.numpy as jnp
from jax.experimental import pallas as pl


def kernel(embeddings, target):
    raise NotImplementedError("write your pallas kernel here")


# speedup vs baseline: 1.4008x; 1.4008x over previous
import jax
import jax.numpy as jnp
from jax.experimental import pallas as pl
from jax.experimental.pallas import tpu as pltpu

_M = 0.25
_GAMMA = 256.0
_B = 4096
_D = 64
_BLK = 256
_NB = _B // _BLK

_OP = 1.0 + _M
_ON = -_M
_DP = 1.0 - _M
_DN = _M


def _circle_loss_kernel(emb_ref, tgt_r_ref, tgt_c_ref, out_ref, embn_ref):
    emb = emb_ref[:, :]
    norm = jnp.sqrt(jnp.sum(emb * emb, axis=1, keepdims=True))
    embn_ref[:, :] = emb / jnp.maximum(norm, 1e-12)

    neg_inf = jnp.float32(-jnp.inf)

    def row_body(i, carry):
        def col_body(j, carry):
            m_p, s_p, m_n, s_n = carry
            rows = embn_ref[pl.ds(i * _BLK, _BLK), :]
            cols = embn_ref[pl.ds(j * _BLK, _BLK), :]
            sim = jax.lax.dot_general(
                rows, cols, (((1,), (1,)), ((), ())),
                preferred_element_type=jnp.float32)
            tr = tgt_r_ref[pl.ds(i * _BLK, _BLK), :]
            tc = tgt_c_ref[:, pl.ds(j * _BLK, _BLK)]
            same = tr == tc
            rid = i * _BLK + jax.lax.broadcasted_iota(
                jnp.int32, (_BLK, _BLK), 0)
            cid = j * _BLK + jax.lax.broadcasted_iota(
                jnp.int32, (_BLK, _BLK), 1)
            triu = cid > rid
            pos = same & triu
            neg = (~same) & triu

            ap = jnp.maximum(_OP - sim, 0.0)
            an = jnp.maximum(sim - _ON, 0.0)
            lp = -_GAMMA * ap * (sim - _DP)
            ln = _GAMMA * an * (sim - _DN)

            bm_p = jnp.max(jnp.where(pos, lp, neg_inf))
            bm_n = jnp.max(jnp.where(neg, ln, neg_inf))
            nm_p = jnp.maximum(m_p, bm_p)
            nm_n = jnp.maximum(m_n, bm_n)

            arg = jnp.where(pos, lp - nm_p, ln - nm_n)
            e = jnp.exp(arg)
            bs_p = jnp.sum(jnp.where(pos, e, 0.0))
            bs_n = jnp.sum(jnp.where(neg, e, 0.0))

            s_p = jnp.where(nm_p == m_p, s_p, s_p * jnp.exp(m_p - nm_p)) + bs_p
            s_n = jnp.where(nm_n == m_n, s_n, s_n * jnp.exp(m_n - nm_n)) + bs_n
            return nm_p, s_p, nm_n, s_n

        return jax.lax.fori_loop(i, _NB, col_body, carry)

    init = (neg_inf, jnp.float32(0.0), neg_inf, jnp.float32(0.0))
    m_p, s_p, m_n, s_n = jax.lax.fori_loop(0, _NB, row_body, init)

    lse_p = m_p + jnp.log(s_p)
    lse_n = m_n + jnp.log(s_n)
    z = lse_p + lse_n
    loss = jnp.maximum(z, 0.0) + jnp.log1p(jnp.exp(-jnp.abs(z)))
    out_ref[0, 0] = loss


@jax.jit
def kernel(embeddings, target):
    tgt_r = target.reshape(_B, 1)
    tgt_c = target.reshape(1, _B)
    out = pl.pallas_call(
        _circle_loss_kernel,
        out_shape=jax.ShapeDtypeStruct((1, 1), jnp.float32),
        out_specs=pl.BlockSpec(memory_space=pltpu.SMEM),
        scratch_shapes=[pltpu.VMEM((_B, _D), jnp.float32)],
    )(embeddings, tgt_r, tgt_c)
    return out[0, 0]


# speedup vs baseline: 1.5430x; 1.1015x over previous
import jax
import jax.numpy as jnp
from jax.experimental import pallas as pl
from jax.experimental.pallas import tpu as pltpu

_M = 0.25
_GAMMA = 256.0
_B = 4096
_D = 64
_BLK = 256
_NB = _B // _BLK

_OP = 1.0 + _M
_ON = -_M
_DP = 1.0 - _M
_DN = _M


def _circle_loss_kernel(emb_ref, tgt_r_ref, tgt_c_ref, out_ref, embn_ref):
    emb = emb_ref[:, :]
    norm = jnp.sqrt(jnp.sum(emb * emb, axis=1, keepdims=True))
    embn_ref[:, :] = emb / jnp.maximum(norm, 1e-12)

    neg_inf = jnp.float32(-jnp.inf)

    def _tile(i, j, diag, carry):
        m_p, s_p, m_n, s_n = carry
        rows = embn_ref[pl.ds(i * _BLK, _BLK), :]
        cols = embn_ref[pl.ds(j * _BLK, _BLK), :]
        sim = jax.lax.dot_general(
            rows, cols, (((1,), (1,)), ((), ())),
            preferred_element_type=jnp.float32)
        tr = tgt_r_ref[pl.ds(i * _BLK, _BLK), :]
        tc = tgt_c_ref[:, pl.ds(j * _BLK, _BLK)]
        same = tr == tc
        if diag:
            rid = jax.lax.broadcasted_iota(jnp.int32, (_BLK, _BLK), 0)
            cid = jax.lax.broadcasted_iota(jnp.int32, (_BLK, _BLK), 1)
            triu = cid > rid
            pos = same & triu
            neg = (~same) & triu
        else:
            pos = same
            neg = ~same

        t = _GAMMA * sim
        lp = (t - _GAMMA * (_OP + _DP)) * sim + _GAMMA * _OP * _DP
        ln = jnp.maximum(sim - _ON, 0.0) * (t - _GAMMA * _DN)

        bm_p = jnp.max(jnp.where(pos, lp, neg_inf))
        bm_n = jnp.max(jnp.where(neg, ln, neg_inf))
        nm_p = jnp.maximum(m_p, bm_p)
        nm_n = jnp.maximum(m_n, bm_n)

        arg = jnp.where(pos, lp - nm_p, ln - nm_n)
        e = jnp.exp(arg)
        bs_p = jnp.sum(jnp.where(pos, e, 0.0))
        if diag:
            bs_n = jnp.sum(jnp.where(neg, e, 0.0))
        else:
            bs_n = jnp.sum(e) - bs_p

        s_p = jnp.where(nm_p == m_p, s_p, s_p * jnp.exp(m_p - nm_p)) + bs_p
        s_n = jnp.where(nm_n == m_n, s_n, s_n * jnp.exp(m_n - nm_n)) + bs_n
        return nm_p, s_p, nm_n, s_n

    init = (neg_inf, jnp.float32(0.0), neg_inf, jnp.float32(0.0))
    carry = jax.lax.fori_loop(
        0, _NB, lambda i, c: _tile(i, i, True, c), init)
    carry = jax.lax.fori_loop(
        0, _NB,
        lambda i, c: jax.lax.fori_loop(
            i + 1, _NB, lambda j, cc: _tile(i, j, False, cc), c),
        carry)
    m_p, s_p, m_n, s_n = carry

    lse_p = m_p + jnp.log(s_p)
    lse_n = m_n + jnp.log(s_n)
    z = lse_p + lse_n
    loss = jnp.maximum(z, 0.0) + jnp.log1p(jnp.exp(-jnp.abs(z)))
    out_ref[0, 0] = loss


@jax.jit
def kernel(embeddings, target):
    tgt_r = target.reshape(_B, 1)
    tgt_c = target.reshape(1, _B)
    out = pl.pallas_call(
        _circle_loss_kernel,
        out_shape=jax.ShapeDtypeStruct((1, 1), jnp.float32),
        out_specs=pl.BlockSpec(memory_space=pltpu.SMEM),
        scratch_shapes=[pltpu.VMEM((_B, _D), jnp.float32)],
    )(embeddings, tgt_r, tgt_c)
    return out[0, 0]


# speedup vs baseline: 1.7806x; 1.1540x over previous
import jax
import jax.numpy as jnp
from jax.experimental import pallas as pl
from jax.experimental.pallas import tpu as pltpu

_M = 0.25
_GAMMA = 256.0
_B = 4096
_D = 64
_BLK = 256
_NB = _B // _BLK

_LOG2E = 1.4426950408889634
_LN2 = 0.6931471805599453
_A = _GAMMA * _LOG2E
_LP_B = -2.0 * _A
_LP_C = 0.9375 * _A
_LN_D = -0.25 * _A


def _fold(x, op):
    r = x
    h = _BLK
    while h > 8:
        h //= 2
        r = op(r[:h], r[h:])
    return r


def _lp2(s):
    return (_A * s + _LP_B) * s + _LP_C


def _ln2(s):
    return jnp.maximum(s + _M, 0.0) * (_A * s + _LN_D)


def _circle_loss_kernel(emb_ref, tgt_r_ref, tgt_c_ref, out_ref, embn_ref):
    emb = emb_ref[:, :]
    norm = jnp.sqrt(jnp.sum(emb * emb, axis=1, keepdims=True))
    embn_ref[:, :] = emb / jnp.maximum(norm, 1e-12)

    rid = jax.lax.broadcasted_iota(jnp.int32, (_BLK, _BLK), 0)
    cid = jax.lax.broadcasted_iota(jnp.int32, (_BLK, _BLK), 1)
    triu = cid > rid
    ntriu = cid <= rid
    neg_inf = jnp.float32(-jnp.inf)

    def _sim_same(i, j):
        rows = embn_ref[pl.ds(i * _BLK, _BLK), :]
        cols = embn_ref[pl.ds(j * _BLK, _BLK), :]
        sim = jax.lax.dot_general(
            rows, cols, (((1,), (1,)), ((), ())),
            preferred_element_type=jnp.float32)
        tr = tgt_r_ref[pl.ds(i * _BLK, _BLK), :]
        tc = tgt_c_ref[:, pl.ds(j * _BLK, _BLK)]
        return sim, tr == tc

    def _ext_tile(i, j, diag, carry):
        mn_p, mn_n, mx_n = carry
        sim, same = _sim_same(i, j)
        if diag:
            pos = same & triu
            notneg = same | ntriu
        else:
            pos = same
            notneg = same
        mn_p = jnp.minimum(mn_p, _fold(jnp.where(pos, sim, 2.0), jnp.minimum))
        mn_n = jnp.minimum(mn_n, _fold(jnp.where(notneg, 2.0, sim),
                                       jnp.minimum))
        mx_n = jnp.maximum(mx_n, _fold(jnp.where(notneg, -2.0, sim),
                                       jnp.maximum))
        return mn_p, mn_n, mx_n

    ext0 = (jnp.full((8, _BLK), 2.0, jnp.float32),
            jnp.full((8, _BLK), 2.0, jnp.float32),
            jnp.full((8, _BLK), -2.0, jnp.float32))
    ext = jax.lax.fori_loop(0, _NB, lambda i, c: _ext_tile(i, i, True, c),
                            ext0)
    ext = jax.lax.fori_loop(
        0, _NB,
        lambda i, c: jax.lax.fori_loop(
            i + 1, _NB, lambda j, cc: _ext_tile(i, j, False, cc), c),
        ext)
    smin_p = jnp.min(ext[0])
    smin_n = jnp.min(ext[1])
    smax_n = jnp.max(ext[2])

    mp2 = _lp2(smin_p)
    mn2 = jnp.maximum(_ln2(smin_n), _ln2(smax_n))
    lp_c = _LP_C - mp2

    def _sum_tile(i, j, diag, carry):
        acc, accp = carry
        sim, same = _sim_same(i, j)
        if diag:
            pos = same & triu
        else:
            pos = same
        lp = (_A * sim + _LP_B) * sim + lp_c
        ln = jnp.maximum(sim + _M, 0.0) * (_A * sim + _LN_D) - mn2
        arg = jnp.where(pos, lp, ln)
        if diag:
            arg = jnp.where(triu, arg, neg_inf)
        e = jnp.exp2(arg)
        acc = acc + _fold(e, jnp.add)
        accp = accp + _fold(jnp.where(pos, e, 0.0), jnp.add)
        return acc, accp

    acc0 = (jnp.zeros((8, _BLK), jnp.float32),
            jnp.zeros((8, _BLK), jnp.float32))
    acc = jax.lax.fori_loop(0, _NB, lambda i, c: _sum_tile(i, i, True, c),
                            acc0)
    acc = jax.lax.fori_loop(
        0, _NB,
        lambda i, c: jax.lax.fori_loop(
            i + 1, _NB, lambda j, cc: _sum_tile(i, j, False, cc), c),
        acc)
    s_p = jnp.sum(acc[1])
    s_n = jnp.sum(acc[0]) - s_p

    lse_p = (mp2 + jnp.log2(s_p)) * _LN2
    lse_n = (mn2 + jnp.log2(s_n)) * _LN2
    z = lse_p + lse_n
    loss = jnp.maximum(z, 0.0) + jnp.log1p(jnp.exp(-jnp.abs(z)))
    out_ref[0, 0] = loss


@jax.jit
def kernel(embeddings, target):
    tgt_r = target.reshape(_B, 1)
    tgt_c = target.reshape(1, _B)
    out = pl.pallas_call(
        _circle_loss_kernel,
        out_shape=jax.ShapeDtypeStruct((1, 1), jnp.float32),
        out_specs=pl.BlockSpec(memory_space=pltpu.SMEM),
        scratch_shapes=[pltpu.VMEM((_B, _D), jnp.float32)],
    )(embeddings, tgt_r, tgt_c)
    return out[0, 0]


# speedup vs baseline: 2.8841x; 1.6197x over previous
import jax
import jax.numpy as jnp
from jax.experimental import pallas as pl
from jax.experimental.pallas import tpu as pltpu

_M = 0.25
_GAMMA = 256.0
_B = 4096
_D = 64
_BLK = 512
_NB = _B // _BLK

_LOG2E = 1.4426950408889634
_LN2 = 0.6931471805599453
_A = _GAMMA * _LOG2E
_LP_B = -2.0 * _A
_LP_C = 0.9375 * _A
_LN_D = -0.25 * _A


def _fold(x, op):
    r = x
    h = _BLK
    while h > 8:
        h //= 2
        r = op(r[:h], r[h:])
    return r


def _lp2(s):
    return (_A * s + _LP_B) * s + _LP_C


def _ln2(s):
    return jnp.maximum(s + _M, 0.0) * (_A * s + _LN_D)


def _circle_loss_kernel(emb_ref, tgt_r_ref, tgt_c_ref, out_ref, embn_ref):
    emb = emb_ref[:, :]
    norm = jnp.sqrt(jnp.sum(emb * emb, axis=1, keepdims=True))
    embn_ref[:, :] = emb / jnp.maximum(norm, 1e-12)

    rid = jax.lax.broadcasted_iota(jnp.int32, (_BLK, _BLK), 0)
    cid = jax.lax.broadcasted_iota(jnp.int32, (_BLK, _BLK), 1)
    triu = cid > rid
    ntriu = cid <= rid
    neg_inf = jnp.float32(-jnp.inf)

    def _sim_same(i, j):
        rows = embn_ref[pl.ds(i * _BLK, _BLK), :]
        cols = embn_ref[pl.ds(j * _BLK, _BLK), :]
        sim = jax.lax.dot_general(
            rows, cols, (((1,), (1,)), ((), ())),
            preferred_element_type=jnp.float32)
        tr = tgt_r_ref[pl.ds(i * _BLK, _BLK), :]
        tc = tgt_c_ref[:, pl.ds(j * _BLK, _BLK)]
        return sim, tr == tc

    def _ext_tile(i, j, diag, carry):
        mn_p, mn_n, mx_n = carry
        sim, same = _sim_same(i, j)
        if diag:
            pos = same & triu
            notneg = same | ntriu
        else:
            pos = same
            notneg = same
        mn_p = jnp.minimum(mn_p, _fold(jnp.where(pos, sim, 2.0), jnp.minimum))
        mn_n = jnp.minimum(mn_n, _fold(jnp.where(notneg, 2.0, sim),
                                       jnp.minimum))
        mx_n = jnp.maximum(mx_n, _fold(jnp.where(notneg, -2.0, sim),
                                       jnp.maximum))
        return mn_p, mn_n, mx_n

    ext0 = (jnp.full((8, _BLK), 2.0, jnp.float32),
            jnp.full((8, _BLK), 2.0, jnp.float32),
            jnp.full((8, _BLK), -2.0, jnp.float32))
    ext = jax.lax.fori_loop(0, _NB, lambda i, c: _ext_tile(i, i, True, c),
                            ext0)
    ext = jax.lax.fori_loop(
        0, _NB,
        lambda i, c: jax.lax.fori_loop(
            i + 1, _NB, lambda j, cc: _ext_tile(i, j, False, cc), c),
        ext)
    smin_p = jnp.min(ext[0])
    smin_n = jnp.min(ext[1])
    smax_n = jnp.max(ext[2])

    mp2 = _lp2(smin_p)
    mn2 = jnp.maximum(_ln2(smin_n), _ln2(smax_n))
    lp_c = _LP_C - mp2

    def _sum_tile(i, j, diag, carry):
        acc, accp = carry
        sim, same = _sim_same(i, j)
        if diag:
            pos = same & triu
        else:
            pos = same
        lp = (_A * sim + _LP_B) * sim + lp_c
        ln = jnp.maximum(sim + _M, 0.0) * (_A * sim + _LN_D) - mn2
        arg = jnp.where(pos, lp, ln)
        if diag:
            arg = jnp.where(triu, arg, neg_inf)
        e = jnp.exp2(arg)
        acc = acc + _fold(e, jnp.add)
        accp = accp + _fold(jnp.where(pos, e, 0.0), jnp.add)
        return acc, accp

    acc0 = (jnp.zeros((8, _BLK), jnp.float32),
            jnp.zeros((8, _BLK), jnp.float32))
    acc = jax.lax.fori_loop(0, _NB, lambda i, c: _sum_tile(i, i, True, c),
                            acc0)
    acc = jax.lax.fori_loop(
        0, _NB,
        lambda i, c: jax.lax.fori_loop(
            i + 1, _NB, lambda j, cc: _sum_tile(i, j, False, cc), c),
        acc)
    s_p = jnp.sum(acc[1])
    s_n = jnp.sum(acc[0]) - s_p

    lse_p = (mp2 + jnp.log2(s_p)) * _LN2
    lse_n = (mn2 + jnp.log2(s_n)) * _LN2
    z = lse_p + lse_n
    loss = jnp.maximum(z, 0.0) + jnp.log1p(jnp.exp(-jnp.abs(z)))
    out_ref[0, 0] = loss


@jax.jit
def kernel(embeddings, target):
    tgt_r = target.reshape(_B, 1)
    tgt_c = target.reshape(1, _B)
    out = pl.pallas_call(
        _circle_loss_kernel,
        out_shape=jax.ShapeDtypeStruct((1, 1), jnp.float32),
        out_specs=pl.BlockSpec(memory_space=pltpu.SMEM),
        scratch_shapes=[pltpu.VMEM((_B, _D), jnp.float32)],
    )(embeddings, tgt_r, tgt_c)
    return out[0, 0]


# speedup vs baseline: 3.4771x; 1.2056x over previous
import jax
import jax.numpy as jnp
from jax.experimental import pallas as pl
from jax.experimental.pallas import tpu as pltpu

_M = 0.25
_GAMMA = 256.0
_B = 4096
_D = 64
_BLK = 1024
_NB = _B // _BLK

_LOG2E = 1.4426950408889634
_LN2 = 0.6931471805599453
_A = _GAMMA * _LOG2E
_LP_B = -2.0 * _A
_LP_C = 0.9375 * _A
_LN_D = -0.25 * _A


def _fold(x, op):
    r = x
    h = _BLK
    while h > 8:
        h //= 2
        r = op(r[:h], r[h:])
    return r


def _lp2(s):
    return (_A * s + _LP_B) * s + _LP_C


def _ln2(s):
    return jnp.maximum(s + _M, 0.0) * (_A * s + _LN_D)


def _circle_loss_kernel(emb_ref, tgt_r_ref, tgt_c_ref, out_ref, embn_ref):
    emb = emb_ref[:, :]
    norm = jnp.sqrt(jnp.sum(emb * emb, axis=1, keepdims=True))
    embn_ref[:, :] = emb / jnp.maximum(norm, 1e-12)

    rid = jax.lax.broadcasted_iota(jnp.int32, (_BLK, _BLK), 0)
    cid = jax.lax.broadcasted_iota(jnp.int32, (_BLK, _BLK), 1)
    triu = cid > rid
    ntriu = cid <= rid
    neg_inf = jnp.float32(-jnp.inf)

    def _sim_same(i, j):
        rows = embn_ref[pl.ds(i * _BLK, _BLK), :]
        cols = embn_ref[pl.ds(j * _BLK, _BLK), :]
        sim = jax.lax.dot_general(
            rows, cols, (((1,), (1,)), ((), ())),
            preferred_element_type=jnp.float32)
        tr = tgt_r_ref[pl.ds(i * _BLK, _BLK), :]
        tc = tgt_c_ref[:, pl.ds(j * _BLK, _BLK)]
        return sim, tr == tc

    def _ext_tile(i, j, diag, carry):
        mn_p, mn_n, mx_n = carry
        sim, same = _sim_same(i, j)
        if diag:
            pos = same & triu
            notneg = same | ntriu
        else:
            pos = same
            notneg = same
        mn_p = jnp.minimum(mn_p, _fold(jnp.where(pos, sim, 2.0), jnp.minimum))
        mn_n = jnp.minimum(mn_n, _fold(jnp.where(notneg, 2.0, sim),
                                       jnp.minimum))
        mx_n = jnp.maximum(mx_n, _fold(jnp.where(notneg, -2.0, sim),
                                       jnp.maximum))
        return mn_p, mn_n, mx_n

    ext0 = (jnp.full((8, _BLK), 2.0, jnp.float32),
            jnp.full((8, _BLK), 2.0, jnp.float32),
            jnp.full((8, _BLK), -2.0, jnp.float32))
    ext = jax.lax.fori_loop(0, _NB, lambda i, c: _ext_tile(i, i, True, c),
                            ext0)
    ext = jax.lax.fori_loop(
        0, _NB,
        lambda i, c: jax.lax.fori_loop(
            i + 1, _NB, lambda j, cc: _ext_tile(i, j, False, cc), c),
        ext)
    smin_p = jnp.min(ext[0])
    smin_n = jnp.min(ext[1])
    smax_n = jnp.max(ext[2])

    mp2 = _lp2(smin_p)
    mn2 = jnp.maximum(_ln2(smin_n), _ln2(smax_n))
    lp_c = _LP_C - mp2

    def _sum_tile(i, j, diag, carry):
        acc, accp = carry
        sim, same = _sim_same(i, j)
        if diag:
            pos = same & triu
        else:
            pos = same
        lp = (_A * sim + _LP_B) * sim + lp_c
        ln = jnp.maximum(sim + _M, 0.0) * (_A * sim + _LN_D) - mn2
        arg = jnp.where(pos, lp, ln)
        if diag:
            arg = jnp.where(triu, arg, neg_inf)
        e = jnp.exp2(arg)
        acc = acc + _fold(e, jnp.add)
        accp = accp + _fold(jnp.where(pos, e, 0.0), jnp.add)
        return acc, accp

    acc0 = (jnp.zeros((8, _BLK), jnp.float32),
            jnp.zeros((8, _BLK), jnp.float32))
    acc = jax.lax.fori_loop(0, _NB, lambda i, c: _sum_tile(i, i, True, c),
                            acc0)
    acc = jax.lax.fori_loop(
        0, _NB,
        lambda i, c: jax.lax.fori_loop(
            i + 1, _NB, lambda j, cc: _sum_tile(i, j, False, cc), c),
        acc)
    s_p = jnp.sum(acc[1])
    s_n = jnp.sum(acc[0]) - s_p

    lse_p = (mp2 + jnp.log2(s_p)) * _LN2
    lse_n = (mn2 + jnp.log2(s_n)) * _LN2
    z = lse_p + lse_n
    loss = jnp.maximum(z, 0.0) + jnp.log1p(jnp.exp(-jnp.abs(z)))
    out_ref[0, 0] = loss


@jax.jit
def kernel(embeddings, target):
    tgt_r = target.reshape(_B, 1)
    tgt_c = target.reshape(1, _B)
    out = pl.pallas_call(
        _circle_loss_kernel,
        out_shape=jax.ShapeDtypeStruct((1, 1), jnp.float32),
        out_specs=pl.BlockSpec(memory_space=pltpu.SMEM),
        scratch_shapes=[pltpu.VMEM((_B, _D), jnp.float32)],
    )(embeddings, tgt_r, tgt_c)
    return out[0, 0]


# speedup vs baseline: 3.8153x; 1.0973x over previous
import jax
import jax.numpy as jnp
from jax.experimental import pallas as pl
from jax.experimental.pallas import tpu as pltpu

_M = 0.25
_GAMMA = 256.0
_B = 4096
_D = 64
_BLK = 1024
_NB = _B // _BLK

_LOG2E = 1.4426950408889634
_LN2 = 0.6931471805599453
_A = _GAMMA * _LOG2E
_LP_B = -2.0 * _A
_LP_C = 0.9375 * _A
_Q_C = -0.0625 * _A


def _fold(x, op):
    r = x
    h = _BLK
    while h > 8:
        h //= 2
        r = op(r[:h], r[h:])
    return r


def _lp2(s):
    return (_A * s + _LP_B) * s + _LP_C


def _ln2(s):
    return jnp.maximum(s + _M, 0.0) * (_A * s - _M * _A)


def _circle_loss_kernel(emb_ref, tgt_r_ref, tgt_c_ref, out_ref, embn_ref):
    emb = emb_ref[:, :]
    norm = jnp.sqrt(jnp.sum(emb * emb, axis=1, keepdims=True))
    embn_ref[:, :] = emb / jnp.maximum(norm, 1e-12)

    rid = jax.lax.broadcasted_iota(jnp.int32, (_BLK, _BLK), 0)
    cid = jax.lax.broadcasted_iota(jnp.int32, (_BLK, _BLK), 1)
    triu = cid > rid
    ondiag = cid == rid
    neg_inf = jnp.float32(-jnp.inf)

    def _sim_same(i, j):
        rows = embn_ref[pl.ds(i * _BLK, _BLK), :]
        cols = embn_ref[pl.ds(j * _BLK, _BLK), :]
        sim = jax.lax.dot_general(
            rows, cols, (((1,), (1,)), ((), ())),
            preferred_element_type=jnp.float32)
        tr = tgt_r_ref[pl.ds(i * _BLK, _BLK), :]
        tc = tgt_c_ref[:, pl.ds(j * _BLK, _BLK)]
        return sim, tr == tc

    def _ext_tile(i, j, diag, carry):
        mn_p, mn_a, mx_a = carry
        sim, same = _sim_same(i, j)
        pos = (same & triu) if diag else same
        mn_p = jnp.minimum(mn_p, _fold(jnp.where(pos, sim, 2.0), jnp.minimum))
        mn_a = jnp.minimum(mn_a, _fold(sim, jnp.minimum))
        mx_sim = jnp.where(ondiag, -2.0, sim) if diag else sim
        mx_a = jnp.maximum(mx_a, _fold(mx_sim, jnp.maximum))
        return mn_p, mn_a, mx_a

    ext0 = (jnp.full((8, _BLK), 2.0, jnp.float32),
            jnp.full((8, _BLK), 2.0, jnp.float32),
            jnp.full((8, _BLK), -2.0, jnp.float32))
    ext = jax.lax.fori_loop(0, _NB, lambda i, c: _ext_tile(i, i, True, c),
                            ext0)
    ext = jax.lax.fori_loop(
        0, _NB,
        lambda i, c: jax.lax.fori_loop(
            i + 1, _NB, lambda j, cc: _ext_tile(i, j, False, cc), c),
        ext)
    smin_p = jnp.min(ext[0])
    smin_a = jnp.min(ext[1])
    smax_a = jnp.max(ext[2])

    mp2 = _lp2(smin_p)
    mn2 = jnp.maximum(_ln2(smin_a), _ln2(smax_a))
    lp_c = _LP_C - mp2
    q_c = _Q_C - mn2
    z_n = -mn2

    def _sum_tile(i, j, diag, carry):
        acc, accp = carry
        sim, same = _sim_same(i, j)
        pos = (same & triu) if diag else same
        t = _A * sim
        lp = (t + _LP_B) * sim + lp_c
        qn = jnp.where(sim < -_M, z_n, t * sim + q_c)
        arg = jnp.where(pos, lp, qn)
        if diag:
            arg = jnp.where(triu, arg, neg_inf)
        e = jnp.exp2(arg)
        acc = acc + _fold(e, jnp.add)
        accp = accp + _fold(jnp.where(pos, e, 0.0), jnp.add)
        return acc, accp

    acc0 = (jnp.zeros((8, _BLK), jnp.float32),
            jnp.zeros((8, _BLK), jnp.float32))
    acc = jax.lax.fori_loop(0, _NB, lambda i, c: _sum_tile(i, i, True, c),
                            acc0)
    acc = jax.lax.fori_loop(
        0, _NB,
        lambda i, c: jax.lax.fori_loop(
            i + 1, _NB, lambda j, cc: _sum_tile(i, j, False, cc), c),
        acc)
    s_p = jnp.sum(acc[1])
    s_n = jnp.sum(acc[0]) - s_p

    lse_p = (mp2 + jnp.log2(s_p)) * _LN2
    lse_n = (mn2 + jnp.log2(s_n)) * _LN2
    z = lse_p + lse_n
    loss = jnp.maximum(z, 0.0) + jnp.log1p(jnp.exp(-jnp.abs(z)))
    out_ref[0, 0] = loss


@jax.jit
def kernel(embeddings, target):
    tgt_r = target.reshape(_B, 1)
    tgt_c = target.reshape(1, _B)
    out = pl.pallas_call(
        _circle_loss_kernel,
        out_shape=jax.ShapeDtypeStruct((1, 1), jnp.float32),
        out_specs=pl.BlockSpec(memory_space=pltpu.SMEM),
        scratch_shapes=[pltpu.VMEM((_B, _D), jnp.float32)],
    )(embeddings, tgt_r, tgt_c)
    return out[0, 0]
